# baseline (speedup 1.0000x reference)
import functools

import jax
import jax.numpy as jnp
from jax import lax
from jax.experimental import pallas as pl
from jax.experimental.pallas import tpu as pltpu
from jax.experimental.pallas import tpu_sc as plsc

_NC, _NS = 2, 16
_D = 128
_S = 32
_NPAD = 10240
_A = 640
_B = 0
_CN = 4
_IDX_PAD = _NPAD * _S + (_A - _B) * _S


def _gather_mean(table, idx_flat):
    mesh = plsc.VectorSubcoreMesh(core_axis_name="c", subcore_axis_name="s")

    @functools.partial(
        pl.kernel,
        out_type=jax.ShapeDtypeStruct((_NPAD, _D), jnp.float32),
        mesh=mesh,
        scratch_types=[
            pltpu.VMEM((_A * _S,), jnp.int32),
            pltpu.VMEM((_CN * _S, _D), jnp.float32),
            pltpu.VMEM((_CN * _S, _D), jnp.float32),
            pltpu.VMEM((2 * _CN, _D), jnp.float32),
            pltpu.SemaphoreType.DMA,
            pltpu.SemaphoreType.DMA,
        ],
    )
    def k(table_hbm, idx_hbm, out_hbm, idx_v, rows0_v, rows1_v, out_v,
          sem0, sem1):
        cid = lax.axis_index("c")
        sid = lax.axis_index("s")
        base = jnp.where(cid == 0, sid * _A, _NS * _A + sid * _B)
        nchunk = jnp.where(cid == 0, _A // _CN, _B // _CN)
        pltpu.sync_copy(idx_hbm.at[pl.ds(base * _S, _A * _S)], idx_v)

        def start(c, rows_v, sem):
            return pltpu.async_copy(
                table_hbm.at[idx_v.at[pl.ds(c * (_CN * _S), _CN * _S)]],
                rows_v, sem)

        def wait(rows_v, sem):
            pltpu.make_async_copy(
                table_hbm.at[idx_v.at[pl.ds(0, _CN * _S)]],
                rows_v, sem).wait()

        def reduce_chunk(slot, rows_v):
            for j in range(_CN):
                def row_body(r, accs):
                    row = j * _S + r * 4
                    for u in range(4):
                        accs = tuple(
                            accs[g] + rows_v[row + u, pl.ds(g * 16, 16)]
                            for g in range(8))
                    return accs
                accs = lax.fori_loop(
                    0, _S // 4, row_body,
                    tuple(jnp.zeros((16,), jnp.float32) for _ in range(8)))
                for g in range(8):
                    out_v[slot * _CN + j, pl.ds(g * 16, 16)] = \
                        accs[g] * (1.0 / _S)

        start(0, rows0_v, sem0)

        def pair_body(t, carry):
            a = t * 2
            start(a + 1, rows1_v, sem1)
            wait(rows0_v, sem0)
            reduce_chunk(0, rows0_v)
            start(jnp.minimum(a + 2, nchunk - 1), rows0_v, sem0)
            wait(rows1_v, sem1)
            reduce_chunk(1, rows1_v)
            pltpu.sync_copy(out_v,
                            out_hbm.at[pl.ds(base + a * _CN, 2 * _CN)])
            return carry

        lax.fori_loop(0, nchunk // 2, pair_body, 0)
        wait(rows0_v, sem0)

    return k(table, idx_flat)


def _sage_linear(a, b, wa, wb, bias, relu):
    npad = a.shape[0]
    bm = 512

    def mm(a_ref, b_ref, wa_ref, wb_ref, bias_ref, o_ref):
        acc = jnp.dot(a_ref[...], wa_ref[...],
                      preferred_element_type=jnp.float32)
        acc = acc + jnp.dot(b_ref[...], wb_ref[...],
                            preferred_element_type=jnp.float32)
        acc = acc + bias_ref[...]
        if relu:
            acc = jnp.maximum(acc, 0.0)
        o_ref[...] = acc

    return pl.pallas_call(
        mm,
        grid=(npad // bm,),
        in_specs=[
            pl.BlockSpec((bm, _D), lambda i: (i, 0)),
            pl.BlockSpec((bm, _D), lambda i: (i, 0)),
            pl.BlockSpec((_D, _D), lambda i: (0, 0)),
            pl.BlockSpec((_D, _D), lambda i: (0, 0)),
            pl.BlockSpec((1, _D), lambda i: (0, 0)),
        ],
        out_specs=pl.BlockSpec((bm, _D), lambda i: (i, 0)),
        out_shape=jax.ShapeDtypeStruct((npad, _D), jnp.float32),
    )(a, b, wa, wb, bias)


def kernel(x, adj, sampled_neighbors, W1, b1, W2, b2):
    n, d = x.shape
    xp = jnp.zeros((_NPAD, d), x.dtype).at[:n].set(x)
    nbrp = jnp.concatenate(
        [sampled_neighbors,
         jnp.zeros((2, _NPAD - n, _S), sampled_neighbors.dtype)], axis=1)
    idx0 = jnp.zeros((_IDX_PAD,), jnp.int32).at[:_NPAD * _S].set(
        nbrp[0].reshape(-1))
    idx1 = jnp.zeros((_IDX_PAD,), jnp.int32).at[:_NPAD * _S].set(
        nbrp[1].reshape(-1))
    w1a, w1b = W1[:d], W1[d:]
    w2a, w2b = W2[:d], W2[d:]

    g1 = _gather_mean(xp, idx0)
    h1 = _sage_linear(xp, g1, w1a, w1b, b1.reshape(1, d), relu=True)
    g2 = _gather_mean(h1, idx1)
    h2 = _sage_linear(h1, g2, w2a, w2b, b2.reshape(1, d), relu=False)
    return h2[:n]


# speedup vs baseline: 1.1962x; 1.1962x over previous
import functools

import jax
import jax.numpy as jnp
from jax import lax
from jax.experimental import pallas as pl
from jax.experimental.pallas import tpu as pltpu
from jax.experimental.pallas import tpu_sc as plsc

_NC, _NS = 2, 16
_D = 128
_S = 32
_NPAD = 10240
_A = 512
_B = 128
_CN = 4
_RING = 4
_IDX_PAD = _NPAD * _S + (_A - _B) * _S


def _gather_mean(table, idx_flat):
    mesh = plsc.VectorSubcoreMesh(core_axis_name="c", subcore_axis_name="s")

    @functools.partial(
        pl.kernel,
        out_type=jax.ShapeDtypeStruct((_NPAD, _D), jnp.float32),
        mesh=mesh,
        scratch_types=[
            pltpu.VMEM((_A * _S,), jnp.int32),
            [pltpu.VMEM((_CN * _S, _D), jnp.float32)
             for _ in range(_RING)],
            [pltpu.VMEM((4 * _CN, _D), jnp.float32)
             for _ in range(2)],
            [pltpu.SemaphoreType.DMA for _ in range(_RING)],
            [pltpu.SemaphoreType.DMA for _ in range(2)],
        ],
    )
    def k(table_hbm, idx_hbm, out_hbm, idx_v, rows, outb, gsem, osem):
        cid = lax.axis_index("c")
        sid = lax.axis_index("s")
        base = jnp.where(cid == 0, sid * _A, _NS * _A + sid * _B)
        nchunk = jnp.where(cid == 0, _A // _CN, _B // _CN)
        nhq = jnp.where(cid == 0, _A // (8 * _CN), _B // (8 * _CN))
        pltpu.sync_copy(idx_hbm.at[pl.ds(base * _S, _A * _S)], idx_v)

        def start(c, rows_v, sem):
            pltpu.async_copy(
                table_hbm.at[idx_v.at[pl.ds(c * (_CN * _S), _CN * _S)]],
                rows_v, sem)

        def wait_gather(k_slot):
            pltpu.make_async_copy(
                table_hbm.at[idx_v.at[pl.ds(0, _CN * _S)]],
                rows[k_slot], gsem[k_slot]).wait()

        def reduce_chunk(rows_v, outb_v, orow):
            for j in range(_CN):
                def row_body(r, accs):
                    row = j * _S + r * 4
                    for u in range(4):
                        accs = tuple(
                            accs[g] + rows_v[row + u, pl.ds(g * 16, 16)]
                            for g in range(8))
                    return accs
                accs = lax.fori_loop(
                    0, _S // 4, row_body,
                    tuple(jnp.zeros((16,), jnp.float32) for _ in range(8)))
                for g in range(8):
                    outb_v[orow + j, pl.ds(g * 16, 16)] = accs[g] * (1.0 / _S)

        for k_slot in range(_RING):
            start(jnp.int32(k_slot), rows[k_slot], gsem[k_slot])

        def hq_body(hq, carry):
            for qi in range(2):
                q = hq * 2 + qi

                @pl.when(hq > 0)
                def _():
                    pltpu.make_async_copy(
                        outb[qi], out_hbm.at[pl.ds(0, 4 * _CN)],
                        osem[qi]).wait()

                for k_slot in range(_RING):
                    c = q * 4 + k_slot
                    wait_gather(k_slot)
                    reduce_chunk(rows[k_slot], outb[qi], k_slot * _CN)
                    start(jnp.minimum(c + _RING, nchunk - 1),
                          rows[k_slot], gsem[k_slot])
                pltpu.async_copy(
                    outb[qi], out_hbm.at[pl.ds(base + q * (4 * _CN), 4 * _CN)],
                    osem[qi])
            return carry

        lax.fori_loop(0, nhq, hq_body, 0)
        for k_slot in range(_RING):
            wait_gather(k_slot)
        for qi in range(2):
            pltpu.make_async_copy(
                outb[qi], out_hbm.at[pl.ds(0, 4 * _CN)], osem[qi]).wait()

    return k(table, idx_flat)


def _sage_linear(a, b, wa, wb, bias, relu):
    npad = a.shape[0]
    bm = 512

    def mm(a_ref, b_ref, wa_ref, wb_ref, bias_ref, o_ref):
        acc = jnp.dot(a_ref[...], wa_ref[...],
                      preferred_element_type=jnp.float32)
        acc = acc + jnp.dot(b_ref[...], wb_ref[...],
                            preferred_element_type=jnp.float32)
        acc = acc + bias_ref[...]
        if relu:
            acc = jnp.maximum(acc, 0.0)
        o_ref[...] = acc

    return pl.pallas_call(
        mm,
        grid=(npad // bm,),
        in_specs=[
            pl.BlockSpec((bm, _D), lambda i: (i, 0)),
            pl.BlockSpec((bm, _D), lambda i: (i, 0)),
            pl.BlockSpec((_D, _D), lambda i: (0, 0)),
            pl.BlockSpec((_D, _D), lambda i: (0, 0)),
            pl.BlockSpec((1, _D), lambda i: (0, 0)),
        ],
        out_specs=pl.BlockSpec((bm, _D), lambda i: (i, 0)),
        out_shape=jax.ShapeDtypeStruct((npad, _D), jnp.float32),
    )(a, b, wa, wb, bias)


def kernel(x, adj, sampled_neighbors, W1, b1, W2, b2):
    n, d = x.shape
    xp = jnp.zeros((_NPAD, d), x.dtype).at[:n].set(x)
    nbrp = jnp.concatenate(
        [sampled_neighbors,
         jnp.zeros((2, _NPAD - n, _S), sampled_neighbors.dtype)], axis=1)
    idx0 = jnp.zeros((_IDX_PAD,), jnp.int32).at[:_NPAD * _S].set(
        nbrp[0].reshape(-1))
    idx1 = jnp.zeros((_IDX_PAD,), jnp.int32).at[:_NPAD * _S].set(
        nbrp[1].reshape(-1))
    w1a, w1b = W1[:d], W1[d:]
    w2a, w2b = W2[:d], W2[d:]

    g1 = _gather_mean(xp, idx0)
    h1 = _sage_linear(xp, g1, w1a, w1b, b1.reshape(1, d), relu=True)
    g2 = _gather_mean(h1, idx1)
    h2 = _sage_linear(h1, g2, w2a, w2b, b2.reshape(1, d), relu=False)
    return h2[:n]


# speedup vs baseline: 1.2195x; 1.0195x over previous
import functools

import jax
import jax.numpy as jnp
from jax import lax
from jax.experimental import pallas as pl
from jax.experimental.pallas import tpu as pltpu
from jax.experimental.pallas import tpu_sc as plsc

_NS = 16
_D = 128
_S = 32
_NPAD = 10240
_NPW = _NPAD // _NS
_CN = 4
_RING = 4
_NCHUNK = _NPW // _CN
_NHQ = _NCHUNK // 8


def _gather_mean(table, idx_flat):
    mesh = plsc.VectorSubcoreMesh(
        core_axis_name="c", subcore_axis_name="s", num_cores=1)

    @functools.partial(
        pl.kernel,
        out_type=jax.ShapeDtypeStruct((_NPAD, _D), jnp.float32),
        mesh=mesh,
        scratch_types=[
            pltpu.VMEM((_NPW * _S,), jnp.int32),
            [pltpu.VMEM((_CN * _S, _D), jnp.float32)
             for _ in range(_RING)],
            [pltpu.VMEM((4 * _CN, _D), jnp.float32)
             for _ in range(2)],
            [pltpu.SemaphoreType.DMA for _ in range(_RING)],
            [pltpu.SemaphoreType.DMA for _ in range(2)],
        ],
    )
    def k(table_hbm, idx_hbm, out_hbm, idx_v, rows, outb, gsem, osem):
        sid = lax.axis_index("s")
        base = sid * _NPW
        pltpu.sync_copy(idx_hbm.at[pl.ds(base * _S, _NPW * _S)], idx_v)

        def start(c, rows_v, sem):
            pltpu.async_copy(
                table_hbm.at[idx_v.at[pl.ds(c * (_CN * _S), _CN * _S)]],
                rows_v, sem)

        def wait_gather(k_slot):
            pltpu.make_async_copy(
                table_hbm.at[idx_v.at[pl.ds(0, _CN * _S)]],
                rows[k_slot], gsem[k_slot]).wait()

        def reduce_chunk(rows_v, outb_v, orow):
            for j in range(_CN):
                def row_body(r, accs):
                    row = j * _S + r * 4
                    for u in range(4):
                        accs = tuple(
                            accs[g] + rows_v[row + u, pl.ds(g * 16, 16)]
                            for g in range(8))
                    return accs
                accs = lax.fori_loop(
                    0, _S // 4, row_body,
                    tuple(jnp.zeros((16,), jnp.float32) for _ in range(8)))
                for g in range(8):
                    outb_v[orow + j, pl.ds(g * 16, 16)] = accs[g] * (1.0 / _S)

        for k_slot in range(_RING):
            start(jnp.int32(k_slot), rows[k_slot], gsem[k_slot])

        def hq_body(hq, carry):
            for qi in range(2):
                q = hq * 2 + qi

                @pl.when(hq > 0)
                def _():
                    pltpu.make_async_copy(
                        outb[qi], out_hbm.at[pl.ds(0, 4 * _CN)],
                        osem[qi]).wait()

                for k_slot in range(_RING):
                    c = q * 4 + k_slot
                    wait_gather(k_slot)
                    reduce_chunk(rows[k_slot], outb[qi], k_slot * _CN)
                    start(jnp.minimum(c + _RING, _NCHUNK - 1),
                          rows[k_slot], gsem[k_slot])
                pltpu.async_copy(
                    outb[qi], out_hbm.at[pl.ds(base + q * (4 * _CN), 4 * _CN)],
                    osem[qi])
            return carry

        lax.fori_loop(0, _NHQ, hq_body, 0)
        for k_slot in range(_RING):
            wait_gather(k_slot)
        for qi in range(2):
            pltpu.make_async_copy(
                outb[qi], out_hbm.at[pl.ds(0, 4 * _CN)], osem[qi]).wait()

    return k(table, idx_flat)


def _sage_linear(a, b, wa, wb, bias, relu):
    npad = a.shape[0]
    bm = 512

    def mm(a_ref, b_ref, wa_ref, wb_ref, bias_ref, o_ref):
        acc = jnp.dot(a_ref[...], wa_ref[...],
                      preferred_element_type=jnp.float32)
        acc = acc + jnp.dot(b_ref[...], wb_ref[...],
                            preferred_element_type=jnp.float32)
        acc = acc + bias_ref[...]
        if relu:
            acc = jnp.maximum(acc, 0.0)
        o_ref[...] = acc

    return pl.pallas_call(
        mm,
        grid=(npad // bm,),
        in_specs=[
            pl.BlockSpec((bm, _D), lambda i: (i, 0)),
            pl.BlockSpec((bm, _D), lambda i: (i, 0)),
            pl.BlockSpec((_D, _D), lambda i: (0, 0)),
            pl.BlockSpec((_D, _D), lambda i: (0, 0)),
            pl.BlockSpec((1, _D), lambda i: (0, 0)),
        ],
        out_specs=pl.BlockSpec((bm, _D), lambda i: (i, 0)),
        out_shape=jax.ShapeDtypeStruct((npad, _D), jnp.float32),
    )(a, b, wa, wb, bias)


def kernel(x, adj, sampled_neighbors, W1, b1, W2, b2):
    n, d = x.shape
    xp = jnp.zeros((_NPAD, d), x.dtype).at[:n].set(x)
    nbrp = jnp.concatenate(
        [sampled_neighbors,
         jnp.zeros((2, _NPAD - n, _S), sampled_neighbors.dtype)], axis=1)
    idx0 = nbrp[0].reshape(-1)
    idx1 = nbrp[1].reshape(-1)
    w1a, w1b = W1[:d], W1[d:]
    w2a, w2b = W2[:d], W2[d:]

    g1 = _gather_mean(xp, idx0)
    h1 = _sage_linear(xp, g1, w1a, w1b, b1.reshape(1, d), relu=True)
    g2 = _gather_mean(h1, idx1)
    h2 = _sage_linear(h1, g2, w2a, w2b, b2.reshape(1, d), relu=False)
    return h2[:n]
